# baseline (speedup 1.0000x reference)
import functools

import jax
import jax.numpy as jnp
from jax import lax
from jax.experimental import pallas as pl
from jax.experimental.pallas import tpu as pltpu
from jax.experimental.pallas import tpu_sc as plsc

B = 16384
EMB = 32
HID = 32
N_USERS = 1000000
N_MOVIES = 100000
NC = 2
NS = 16
NW = NC * NS
PK = 128 // EMB
NLINES = B // PK
RPW = NLINES // NW


def _sc_gather(uw_idx, mw_idx, ut_flat, mt_flat):
    mesh = plsc.VectorSubcoreMesh(core_axis_name="c", subcore_axis_name="s")

    @functools.partial(
        pl.kernel,
        mesh=mesh,
        compiler_params=pltpu.CompilerParams(use_tc_tiling_on_sc=False),
        out_type=(
            jax.ShapeDtypeStruct((NLINES, 128), jnp.float32),
            jax.ShapeDtypeStruct((NLINES, 128), jnp.float32),
        ),
        scratch_types=[
            pltpu.VMEM((RPW, 128), jnp.int32),
            pltpu.VMEM((RPW, 128), jnp.int32),
            pltpu.VMEM((RPW, 128), jnp.float32),
            pltpu.VMEM((RPW, 128), jnp.float32),
            pltpu.SemaphoreType.DMA,
            pltpu.SemaphoreType.DMA,
        ],
    )
    def k(uw_hbm, mw_hbm, ut_hbm, mt_hbm, u_out, m_out,
          uw_v, mw_v, upk_v, mpk_v, sem_u, sem_m):
        wid = lax.axis_index("s") * NC + lax.axis_index("c")
        base = wid * RPW
        pltpu.sync_copy(uw_hbm.at[pl.ds(base, RPW)], uw_v)
        pltpu.sync_copy(mw_hbm.at[pl.ds(base, RPW)], mw_v)

        def issue(j, _):
            pltpu.async_copy(ut_hbm.at[uw_v.at[j]], upk_v.at[j], sem_u)
            pltpu.async_copy(mt_hbm.at[mw_v.at[j]], mpk_v.at[j], sem_m)
            return 0

        lax.fori_loop(0, RPW, issue, 0)
        pltpu.make_async_copy(u_out.at[pl.ds(0, RPW)], upk_v, sem_u).wait()
        pltpu.make_async_copy(m_out.at[pl.ds(0, RPW)], mpk_v, sem_m).wait()
        pltpu.sync_copy(upk_v, u_out.at[pl.ds(base, RPW)])
        pltpu.sync_copy(mpk_v, m_out.at[pl.ds(base, RPW)])

    return k(uw_idx, mw_idx, ut_flat, mt_flat)


def _tc_mlp(u4, m4, w1u_bd, w1m_bd, b1_t, w2_bd, b2_2d):
    BLK = 512

    def body(u_ref, m_ref, w1u_ref, w1m_ref, b1_ref, w2_ref, b2_ref, o_ref):
        h = jnp.dot(u_ref[...], w1u_ref[...], preferred_element_type=jnp.float32)
        h = h + jnp.dot(m_ref[...], w1m_ref[...], preferred_element_type=jnp.float32)
        h = jnp.maximum(h + b1_ref[...], 0.0)
        o = jnp.dot(h, w2_ref[...], preferred_element_type=jnp.float32) + b2_ref[0, 0]
        o_ref[...] = jnp.maximum(o, 0.0)

    return pl.pallas_call(
        body,
        grid=(NLINES // BLK,),
        in_specs=[
            pl.BlockSpec((BLK, 128), lambda i: (i, 0)),
            pl.BlockSpec((BLK, 128), lambda i: (i, 0)),
            pl.BlockSpec((128, 128), lambda i: (0, 0)),
            pl.BlockSpec((128, 128), lambda i: (0, 0)),
            pl.BlockSpec((1, 128), lambda i: (0, 0)),
            pl.BlockSpec((128, PK), lambda i: (0, 0)),
            pl.BlockSpec((1, 1), lambda i: (0, 0)),
        ],
        out_specs=pl.BlockSpec((BLK, PK), lambda i: (i, 0)),
        out_shape=jax.ShapeDtypeStruct((NLINES, PK), jnp.float32),
    )(u4, m4, w1u_bd, w1m_bd, b1_t, w2_bd, b2_2d)


def _word_indices(idx, n_rows):
    i4 = idx.astype(jnp.int32).reshape(NLINES, PK, 1)
    offs = (jnp.arange(EMB, dtype=jnp.int32) * n_rows).reshape(1, 1, EMB)
    return (i4 + offs).reshape(NLINES, 128)


def kernel(u_idx, m_idx, user_table, movie_table, W1, b1, W2, b2):
    uw_idx = _word_indices(u_idx, N_USERS)
    mw_idx = _word_indices(m_idx, N_MOVIES)
    ut_flat = user_table.T.reshape(-1)
    mt_flat = movie_table.T.reshape(-1)
    u4, m4 = _sc_gather(uw_idx, mw_idx, ut_flat, mt_flat)
    eye = jnp.eye(PK, dtype=jnp.float32)
    w1u_bd = jnp.kron(eye, W1[:, :EMB].T)
    w1m_bd = jnp.kron(eye, W1[:, EMB:].T)
    w2_bd = jnp.kron(eye, W2.T)
    b1_t = jnp.tile(b1, PK).reshape(1, 128)
    out4 = _tc_mlp(u4, m4, w1u_bd, w1m_bd, b1_t, w2_bd, b2.reshape(1, 1))
    return out4.reshape(B)


# speedup vs baseline: 4.9941x; 4.9941x over previous
import functools

import jax
import jax.numpy as jnp
from jax import lax
from jax.experimental import pallas as pl
from jax.experimental.pallas import tpu as pltpu
from jax.experimental.pallas import tpu_sc as plsc

B = 16384
EMB = 32
HID = 32
N_USERS = 1000000
N_MOVIES = 100000
NC = 2
NS = 16
NW = NC * NS
BPW = B // NW
CHUNK = 128
NCHUNK = BPW // CHUNK


def _tc_transpose_pad(table_t, n_rows):
    BLKC = 2048

    def body(x_ref, o_ref):
        o_ref[:, :EMB] = x_ref[...].T

    return pl.pallas_call(
        body,
        grid=(pl.cdiv(n_rows, BLKC),),
        in_specs=[pl.BlockSpec((EMB, BLKC), lambda i: (0, i))],
        out_specs=pl.BlockSpec((BLKC, 128), lambda i: (i, 0)),
        out_shape=jax.ShapeDtypeStruct((n_rows, 128), jnp.float32),
    )(table_t)


def _sc_gather(u_idx2d, m_idx2d, ut_p, mt_p):
    mesh = plsc.VectorSubcoreMesh(core_axis_name="c", subcore_axis_name="s")

    @functools.partial(
        pl.kernel,
        mesh=mesh,
        out_type=(
            jax.ShapeDtypeStruct((B, 128), jnp.float32),
            jax.ShapeDtypeStruct((B, 128), jnp.float32),
        ),
        scratch_types=[
            pltpu.VMEM((NCHUNK, CHUNK), jnp.int32),
            pltpu.VMEM((NCHUNK, CHUNK), jnp.int32),
            pltpu.VMEM((CHUNK, 128), jnp.float32),
            pltpu.VMEM((CHUNK, 128), jnp.float32),
            pltpu.SemaphoreType.DMA,
            pltpu.SemaphoreType.DMA,
        ],
    )
    def k(u_idx_hbm, m_idx_hbm, ut_hbm, mt_hbm, u_out, m_out,
          uidx_v, midx_v, ubuf_v, mbuf_v, sem_u, sem_m):
        wid = lax.axis_index("s") * NC + lax.axis_index("c")
        base = wid * BPW
        pltpu.sync_copy(u_idx_hbm.at[pl.ds(wid * NCHUNK, NCHUNK)], uidx_v)
        pltpu.sync_copy(m_idx_hbm.at[pl.ds(wid * NCHUNK, NCHUNK)], midx_v)
        for j in range(NCHUNK):
            cu = pltpu.async_copy(ut_hbm.at[uidx_v.at[j]], ubuf_v, sem_u)
            cm = pltpu.async_copy(mt_hbm.at[midx_v.at[j]], mbuf_v, sem_m)
            cu.wait()
            pltpu.sync_copy(ubuf_v, u_out.at[pl.ds(base + j * CHUNK, CHUNK)])
            cm.wait()
            pltpu.sync_copy(mbuf_v, m_out.at[pl.ds(base + j * CHUNK, CHUNK)])

    return k(u_idx2d, m_idx2d, ut_p, mt_p)


def _tc_mlp(u_rows, m_rows, w1u_t, w1m_t, b1_2d, w2_2d, b2_2d):
    BLK = 2048

    def body(u_ref, m_ref, w1u_ref, w1m_ref, b1_ref, w2_ref, b2_ref, o_ref):
        xu = u_ref[:, :EMB]
        xm = m_ref[:, :EMB]
        h = jnp.dot(xu, w1u_ref[...], preferred_element_type=jnp.float32)
        h = h + jnp.dot(xm, w1m_ref[...], preferred_element_type=jnp.float32)
        h = jnp.maximum(h + b1_ref[...], 0.0)
        o = jnp.sum(h * w2_ref[...], axis=1, keepdims=True) + b2_ref[0, 0]
        o_ref[...] = jnp.maximum(o, 0.0)

    out = pl.pallas_call(
        body,
        grid=(B // BLK,),
        in_specs=[
            pl.BlockSpec((BLK, 128), lambda i: (i, 0)),
            pl.BlockSpec((BLK, 128), lambda i: (i, 0)),
            pl.BlockSpec((EMB, HID), lambda i: (0, 0)),
            pl.BlockSpec((EMB, HID), lambda i: (0, 0)),
            pl.BlockSpec((1, HID), lambda i: (0, 0)),
            pl.BlockSpec((1, HID), lambda i: (0, 0)),
            pl.BlockSpec((1, 1), lambda i: (0, 0)),
        ],
        out_specs=pl.BlockSpec((BLK, 1), lambda i: (i, 0)),
        out_shape=jax.ShapeDtypeStruct((B, 1), jnp.float32),
    )(u_rows, m_rows, w1u_t, w1m_t, b1_2d, w2_2d, b2_2d)
    return out[:, 0]


def kernel(u_idx, m_idx, user_table, movie_table, W1, b1, W2, b2):
    u_idx2d = u_idx.astype(jnp.int32).reshape(B // CHUNK, CHUNK)
    m_idx2d = m_idx.astype(jnp.int32).reshape(B // CHUNK, CHUNK)
    ut_p = _tc_transpose_pad(user_table.T, N_USERS)
    mt_p = _tc_transpose_pad(movie_table.T, N_MOVIES)
    u_rows, m_rows = _sc_gather(u_idx2d, m_idx2d, ut_p, mt_p)
    w1u_t = W1[:, :EMB].T
    w1m_t = W1[:, EMB:].T
    return _tc_mlp(u_rows, m_rows, w1u_t, w1m_t,
                   b1.reshape(1, HID), W2, b2.reshape(1, 1))


# speedup vs baseline: 5.0715x; 1.0155x over previous
import functools

import jax
import jax.numpy as jnp
from jax import lax
from jax.experimental import pallas as pl
from jax.experimental.pallas import tpu as pltpu
from jax.experimental.pallas import tpu_sc as plsc

B = 16384
EMB = 32
HID = 32
N_USERS = 1000000
N_MOVIES = 100000
NC = 2
NS = 16
NW = NC * NS
BPW = B // NW
CHUNK = 128
NCHUNK = BPW // CHUNK
BLKC = 2048
GRP = BLKC // 4


def _tc_transpose_pack(table_t, n_rows):
    nblk = (n_rows + BLKC - 1) // BLKC

    def body(x_ref, o_ref):
        for k in range(4):
            o_ref[:, k * EMB:(k + 1) * EMB] = x_ref[:, k * GRP:(k + 1) * GRP].T

    return pl.pallas_call(
        body,
        grid=(nblk,),
        in_specs=[pl.BlockSpec((EMB, BLKC), lambda i: (0, i))],
        out_specs=pl.BlockSpec((GRP, 128), lambda i: (i, 0)),
        out_shape=jax.ShapeDtypeStruct((nblk * GRP, 128), jnp.float32),
    )(table_t)


def _sc_gather(u_idx2d, m_idx2d, ut_p, mt_p):
    mesh = plsc.VectorSubcoreMesh(core_axis_name="c", subcore_axis_name="s")

    @functools.partial(
        pl.kernel,
        mesh=mesh,
        out_type=(
            jax.ShapeDtypeStruct((B, 128), jnp.float32),
            jax.ShapeDtypeStruct((B, 128), jnp.float32),
        ),
        scratch_types=[
            pltpu.VMEM((NCHUNK, CHUNK), jnp.int32),
            pltpu.VMEM((NCHUNK, CHUNK), jnp.int32),
            pltpu.VMEM((NCHUNK, CHUNK), jnp.int32),
            pltpu.VMEM((NCHUNK, CHUNK), jnp.int32),
            pltpu.VMEM((CHUNK, 128), jnp.float32),
            pltpu.VMEM((CHUNK, 128), jnp.float32),
            pltpu.SemaphoreType.DMA,
            pltpu.SemaphoreType.DMA,
        ],
    )
    def k(u_idx_hbm, m_idx_hbm, ut_hbm, mt_hbm, u_out, m_out,
          uidx_v, midx_v, uln_v, mln_v, ubuf_v, mbuf_v, sem_u, sem_m):
        wid = lax.axis_index("s") * NC + lax.axis_index("c")
        base = wid * BPW
        pltpu.sync_copy(u_idx_hbm.at[pl.ds(wid * NCHUNK, NCHUNK)], uidx_v)
        pltpu.sync_copy(m_idx_hbm.at[pl.ds(wid * NCHUNK, NCHUNK)], midx_v)
        L = 16
        for j in range(NCHUNK):
            for g in range(CHUNK // L):
                sl = pl.ds(g * L, L)
                uv = uidx_v[j, sl]
                mv = midx_v[j, sl]
                uln_v[j, sl] = (lax.shift_left(
                    lax.shift_right_logical(uv, 11), 9)
                    + lax.bitwise_and(uv, GRP - 1))
                mln_v[j, sl] = (lax.shift_left(
                    lax.shift_right_logical(mv, 11), 9)
                    + lax.bitwise_and(mv, GRP - 1))
        for j in range(NCHUNK):
            cu = pltpu.async_copy(ut_hbm.at[uln_v.at[j]], ubuf_v, sem_u)
            cm = pltpu.async_copy(mt_hbm.at[mln_v.at[j]], mbuf_v, sem_m)
            cu.wait()
            pltpu.sync_copy(ubuf_v, u_out.at[pl.ds(base + j * CHUNK, CHUNK)])
            cm.wait()
            pltpu.sync_copy(mbuf_v, m_out.at[pl.ds(base + j * CHUNK, CHUNK)])

    return k(u_idx2d, m_idx2d, ut_p, mt_p)


def _tc_mlp(u_rows, m_rows, ku, km, w1u_t, w1m_t, b1_2d, w2_2d, b2_2d):
    BLK = 2048

    def body(u_ref, m_ref, ku_ref, km_ref, w1u_ref, w1m_ref, b1_ref,
             w2_ref, b2_ref, o_ref):
        xu = jnp.zeros((BLK, EMB), jnp.float32)
        xm = jnp.zeros((BLK, EMB), jnp.float32)
        kub = ku_ref[...]
        kmb = km_ref[...]
        for k in range(4):
            su = (kub == k).astype(jnp.float32)
            sm = (kmb == k).astype(jnp.float32)
            xu = xu + su * u_ref[:, k * EMB:(k + 1) * EMB]
            xm = xm + sm * m_ref[:, k * EMB:(k + 1) * EMB]
        h = jnp.dot(xu, w1u_ref[...], preferred_element_type=jnp.float32)
        h = h + jnp.dot(xm, w1m_ref[...], preferred_element_type=jnp.float32)
        h = jnp.maximum(h + b1_ref[...], 0.0)
        o = jnp.sum(h * w2_ref[...], axis=1, keepdims=True) + b2_ref[0, 0]
        o_ref[...] = jnp.maximum(o, 0.0)

    out = pl.pallas_call(
        body,
        grid=(B // BLK,),
        in_specs=[
            pl.BlockSpec((BLK, 128), lambda i: (i, 0)),
            pl.BlockSpec((BLK, 128), lambda i: (i, 0)),
            pl.BlockSpec((BLK, 1), lambda i: (i, 0)),
            pl.BlockSpec((BLK, 1), lambda i: (i, 0)),
            pl.BlockSpec((EMB, HID), lambda i: (0, 0)),
            pl.BlockSpec((EMB, HID), lambda i: (0, 0)),
            pl.BlockSpec((1, HID), lambda i: (0, 0)),
            pl.BlockSpec((1, HID), lambda i: (0, 0)),
            pl.BlockSpec((1, 1), lambda i: (0, 0)),
        ],
        out_specs=pl.BlockSpec((BLK, 1), lambda i: (i, 0)),
        out_shape=jax.ShapeDtypeStruct((B, 1), jnp.float32),
    )(u_rows, m_rows, ku, km, w1u_t, w1m_t, b1_2d, w2_2d, b2_2d)
    return out[:, 0]


def kernel(u_idx, m_idx, user_table, movie_table, W1, b1, W2, b2):
    u32 = u_idx.astype(jnp.int32)
    m32 = m_idx.astype(jnp.int32)
    u_idx2d = u32.reshape(B // CHUNK, CHUNK)
    m_idx2d = m32.reshape(B // CHUNK, CHUNK)
    ut_p = _tc_transpose_pack(user_table.T, N_USERS)
    mt_p = _tc_transpose_pack(movie_table.T, N_MOVIES)
    u_rows, m_rows = _sc_gather(u_idx2d, m_idx2d, ut_p, mt_p)
    ku = lax.bitwise_and(lax.shift_right_logical(u32, 9), 3).reshape(B, 1)
    km = lax.bitwise_and(lax.shift_right_logical(m32, 9), 3).reshape(B, 1)
    w1u_t = W1[:, :EMB].T
    w1m_t = W1[:, EMB:].T
    return _tc_mlp(u_rows, m_rows, ku, km, w1u_t, w1m_t,
                   b1.reshape(1, HID), W2, b2.reshape(1, 1))


# speedup vs baseline: 7.6434x; 1.5071x over previous
import functools

import jax
import jax.numpy as jnp
from jax import lax
from jax.experimental import pallas as pl
from jax.experimental.pallas import tpu as pltpu
from jax.experimental.pallas import tpu_sc as plsc

B = 16384
EMB = 32
HID = 32
N_USERS = 1000000
N_MOVIES = 100000
NC = 2
NS = 16
NW = NC * NS
BPW = B // NW
CHUNK = 128
NCHUNK = BPW // CHUNK
BLKC = 8192
GRP = BLKC // 4


def _tc_transpose_pack(table_t, n_rows):
    nblk = (n_rows + BLKC - 1) // BLKC

    def body(x_ref, o_ref):
        for k in range(4):
            o_ref[:, k * EMB:(k + 1) * EMB] = x_ref[:, k * GRP:(k + 1) * GRP].T

    return pl.pallas_call(
        body,
        grid=(nblk,),
        in_specs=[pl.BlockSpec((EMB, BLKC), lambda i: (0, i))],
        out_specs=pl.BlockSpec((GRP, 128), lambda i: (i, 0)),
        out_shape=jax.ShapeDtypeStruct((nblk * GRP, 128), jnp.float32),
    )(table_t)


def _sc_gather(u_idx2d, m_idx2d, ut_p, mt_p):
    mesh = plsc.VectorSubcoreMesh(core_axis_name="c", subcore_axis_name="s")

    @functools.partial(
        pl.kernel,
        mesh=mesh,
        out_type=(
            jax.ShapeDtypeStruct((B, 128), jnp.float32),
            jax.ShapeDtypeStruct((B, 128), jnp.float32),
        ),
        scratch_types=[
            pltpu.VMEM((NCHUNK, CHUNK), jnp.int32),
            pltpu.VMEM((NCHUNK, CHUNK), jnp.int32),
            pltpu.VMEM((NCHUNK, CHUNK), jnp.int32),
            pltpu.VMEM((NCHUNK, CHUNK), jnp.int32),
            pltpu.VMEM((CHUNK, 128), jnp.float32),
            pltpu.VMEM((CHUNK, 128), jnp.float32),
            pltpu.SemaphoreType.DMA,
            pltpu.SemaphoreType.DMA,
        ],
    )
    def k(u_idx_hbm, m_idx_hbm, ut_hbm, mt_hbm, u_out, m_out,
          uidx_v, midx_v, uln_v, mln_v, ubuf_v, mbuf_v, sem_u, sem_m):
        wid = lax.axis_index("s") * NC + lax.axis_index("c")
        base = wid * BPW
        pltpu.sync_copy(u_idx_hbm.at[pl.ds(wid * NCHUNK, NCHUNK)], uidx_v)
        pltpu.sync_copy(m_idx_hbm.at[pl.ds(wid * NCHUNK, NCHUNK)], midx_v)
        L = 16
        for j in range(NCHUNK):
            for g in range(CHUNK // L):
                sl = pl.ds(g * L, L)
                uv = uidx_v[j, sl]
                mv = midx_v[j, sl]
                uln_v[j, sl] = (lax.shift_left(
                    lax.shift_right_logical(uv, 13), 11)
                    + lax.bitwise_and(uv, GRP - 1))
                mln_v[j, sl] = (lax.shift_left(
                    lax.shift_right_logical(mv, 13), 11)
                    + lax.bitwise_and(mv, GRP - 1))
        for j in range(NCHUNK):
            cu = pltpu.async_copy(ut_hbm.at[uln_v.at[j]], ubuf_v, sem_u)
            cm = pltpu.async_copy(mt_hbm.at[mln_v.at[j]], mbuf_v, sem_m)
            cu.wait()
            pltpu.sync_copy(ubuf_v, u_out.at[pl.ds(base + j * CHUNK, CHUNK)])
            cm.wait()
            pltpu.sync_copy(mbuf_v, m_out.at[pl.ds(base + j * CHUNK, CHUNK)])

    return k(u_idx2d, m_idx2d, ut_p, mt_p)


def _tc_mlp(u_rows, m_rows, ku, km, w1u_t, w1m_t, b1_2d, w2_2d, b2_2d):
    BLK = 2048

    def body(u_ref, m_ref, ku_ref, km_ref, w1u_ref, w1m_ref, b1_ref,
             w2_ref, b2_ref, o_ref):
        xu = jnp.zeros((BLK, EMB), jnp.float32)
        xm = jnp.zeros((BLK, EMB), jnp.float32)
        kub = ku_ref[...]
        kmb = km_ref[...]
        for k in range(4):
            su = (kub == k).astype(jnp.float32)
            sm = (kmb == k).astype(jnp.float32)
            xu = xu + su * u_ref[:, k * EMB:(k + 1) * EMB]
            xm = xm + sm * m_ref[:, k * EMB:(k + 1) * EMB]
        h = jnp.dot(xu, w1u_ref[...], preferred_element_type=jnp.float32)
        h = h + jnp.dot(xm, w1m_ref[...], preferred_element_type=jnp.float32)
        h = jnp.maximum(h + b1_ref[...], 0.0)
        o = jnp.sum(h * w2_ref[...], axis=1, keepdims=True) + b2_ref[0, 0]
        o_ref[...] = jnp.maximum(o, 0.0)

    out = pl.pallas_call(
        body,
        grid=(B // BLK,),
        in_specs=[
            pl.BlockSpec((BLK, 128), lambda i: (i, 0)),
            pl.BlockSpec((BLK, 128), lambda i: (i, 0)),
            pl.BlockSpec((BLK, 1), lambda i: (i, 0)),
            pl.BlockSpec((BLK, 1), lambda i: (i, 0)),
            pl.BlockSpec((EMB, HID), lambda i: (0, 0)),
            pl.BlockSpec((EMB, HID), lambda i: (0, 0)),
            pl.BlockSpec((1, HID), lambda i: (0, 0)),
            pl.BlockSpec((1, HID), lambda i: (0, 0)),
            pl.BlockSpec((1, 1), lambda i: (0, 0)),
        ],
        out_specs=pl.BlockSpec((BLK, 1), lambda i: (i, 0)),
        out_shape=jax.ShapeDtypeStruct((B, 1), jnp.float32),
    )(u_rows, m_rows, ku, km, w1u_t, w1m_t, b1_2d, w2_2d, b2_2d)
    return out[:, 0]


def kernel(u_idx, m_idx, user_table, movie_table, W1, b1, W2, b2):
    u32 = u_idx.astype(jnp.int32)
    m32 = m_idx.astype(jnp.int32)
    u_idx2d = u32.reshape(B // CHUNK, CHUNK)
    m_idx2d = m32.reshape(B // CHUNK, CHUNK)
    ut_p = _tc_transpose_pack(user_table.T, N_USERS)
    mt_p = _tc_transpose_pack(movie_table.T, N_MOVIES)
    u_rows, m_rows = _sc_gather(u_idx2d, m_idx2d, ut_p, mt_p)
    ku = lax.bitwise_and(lax.shift_right_logical(u32, 11), 3).reshape(B, 1)
    km = lax.bitwise_and(lax.shift_right_logical(m32, 11), 3).reshape(B, 1)
    w1u_t = W1[:, :EMB].T
    w1m_t = W1[:, EMB:].T
    return _tc_mlp(u_rows, m_rows, ku, km, w1u_t, w1m_t,
                   b1.reshape(1, HID), W2, b2.reshape(1, 1))
